# baseline (speedup 1.0000x reference)
import jax
import jax.numpy as jnp
from jax import lax
from jax.experimental import pallas as pl
from jax.experimental.pallas import tpu as pltpu
from jax.experimental.pallas import tpu_sc as plsc

_INFO = plsc.get_sparse_core_info()
_NC, _NS, _L = _INFO.num_cores, _INFO.num_subcores, _INFO.num_lanes
_NW = _NC * _NS

_B = 16384
_D = 64
_BPW = _B // _NW
_GCHUNK = 128
_NG = _BPW // _GCHUNK


def _gmf_body(upidx_hbm, ipidx_hbm, uraw_hbm, iraw_hbm, utab2_hbm, itab2_hbm,
              w_hbm, b_hbm,
              out_hbm,
              upidx_v, ipidx_v, uraw_v, iraw_v, ubuf, ibuf, w_v, b_v,
              accbuf, outbuf, sem):
    wid = lax.axis_index("s") * _NC + lax.axis_index("c")
    base = wid * _BPW

    pltpu.sync_copy(upidx_hbm.at[wid], upidx_v)
    pltpu.sync_copy(ipidx_hbm.at[wid], ipidx_v)
    pltpu.sync_copy(uraw_hbm.at[wid], uraw_v)
    pltpu.sync_copy(iraw_hbm.at[wid], iraw_v)
    pltpu.sync_copy(w_hbm, w_v)
    pltpu.sync_copy(b_hbm, b_v)

    w0 = w_v[pl.ds(0, _L)]
    w1 = w_v[pl.ds(_L, _L)]
    w2 = w_v[pl.ds(2 * _L, _L)]
    w3 = w_v[pl.ds(3 * _L, _L)]
    bias = b_v[...]

    def fire(ch, slot):
        pltpu.async_copy(utab2_hbm.at[upidx_v.at[ch]], ubuf.at[slot], sem)
        pltpu.async_copy(itab2_hbm.at[ipidx_v.at[ch]], ibuf.at[slot], sem)

    def drain(slot):
        pltpu.make_async_copy(utab2_hbm.at[pl.ds(0, _GCHUNK)],
                              ubuf.at[slot], sem).wait()
        pltpu.make_async_copy(utab2_hbm.at[pl.ds(0, _GCHUNK)],
                              ibuf.at[slot], sem).wait()

    fire(0, 0)

    def chunk_body(ch, carry):
        slot = lax.rem(ch, 2)

        @pl.when(ch + 1 < _NG)
        def _():
            fire(ch + 1, 1 - slot)

        drain(slot)

        def grp16_body(k16, c2):
            rbase = ch * _GCHUNK + k16 * _L
            uoff_v = (uraw_v[pl.ds(rbase, _L)] & 1) * _D
            ioff_v = (iraw_v[pl.ds(rbase, _L)] & 1) * _D
            for k in range(_L):
                e = k16 * _L + k
                uoff = uoff_v[k]
                ioff = ioff_v[k]
                acc = (ubuf[slot, e, pl.ds(uoff, _L)]
                       * ibuf[slot, e, pl.ds(ioff, _L)] * w0
                       + ubuf[slot, e, pl.ds(uoff + _L, _L)]
                       * ibuf[slot, e, pl.ds(ioff + _L, _L)] * w1
                       + ubuf[slot, e, pl.ds(uoff + 2 * _L, _L)]
                       * ibuf[slot, e, pl.ds(ioff + 2 * _L, _L)] * w2
                       + ubuf[slot, e, pl.ds(uoff + 3 * _L, _L)]
                       * ibuf[slot, e, pl.ds(ioff + 3 * _L, _L)] * w3)
                accbuf[pl.ds((rbase + k) * 17, _L)] = acc
            return c2

        lax.fori_loop(0, _GCHUNK // _L, grp16_body, 0)
        return carry

    lax.fori_loop(0, _NG, chunk_body, 0)

    iota16 = lax.iota(jnp.int32, _L)

    def grp_body(g, carry):
        flat = iota16 * 17 + g * (_L * 17)
        acc = bias
        for l in range(_L):
            acc = acc + plsc.load_gather(accbuf, [flat + l])
        outbuf[pl.ds(g * _L, _L)] = 1.0 / (1.0 + jnp.exp(-acc))
        return carry

    lax.fori_loop(0, _BPW // _L, grp_body, 0)

    pltpu.sync_copy(outbuf, out_hbm.at[pl.ds(base, _BPW)])


@jax.jit
def _gmf_call(upidx, ipidx, uraw, iraw, utab2, itab2, w_flat, b_vec):
    mesh = plsc.VectorSubcoreMesh(core_axis_name="c", subcore_axis_name="s")
    return pl.kernel(
        _gmf_body,
        mesh=mesh,
        out_type=jax.ShapeDtypeStruct((_B,), jnp.float32),
        scratch_types=[
            pltpu.VMEM((_NG, _GCHUNK), jnp.int32),
            pltpu.VMEM((_NG, _GCHUNK), jnp.int32),
            pltpu.VMEM((_BPW,), jnp.int32),
            pltpu.VMEM((_BPW,), jnp.int32),
            pltpu.VMEM((2, _GCHUNK, 2 * _D), jnp.float32),
            pltpu.VMEM((2, _GCHUNK, 2 * _D), jnp.float32),
            pltpu.VMEM((_D,), jnp.float32),
            pltpu.VMEM((_L,), jnp.float32),
            pltpu.VMEM((_BPW * 17,), jnp.float32),
            pltpu.VMEM((_BPW,), jnp.float32),
            pltpu.SemaphoreType.DMA,
        ],
        compiler_params=pltpu.CompilerParams(needs_layout_passes=False),
    )(upidx, ipidx, uraw, iraw, utab2, itab2, w_flat, b_vec)


def kernel(user_indices, item_indices, user_table, item_table, affine_w,
           affine_b):
    uidx = user_indices.astype(jnp.int32)
    iidx = item_indices.astype(jnp.int32)
    upidx = (uidx >> 1).reshape(_NW, _NG, _GCHUNK)
    ipidx = (iidx >> 1).reshape(_NW, _NG, _GCHUNK)
    uraw = uidx.reshape(_NW, _BPW)
    iraw = iidx.reshape(_NW, _BPW)
    utab2 = user_table.reshape(500000, 2 * _D)
    itab2 = item_table.reshape(500000, 2 * _D)
    w_flat = affine_w.reshape(_D)
    b_vec = jnp.broadcast_to(affine_b.reshape(()), (_L,))
    out = _gmf_call(upidx, ipidx, uraw, iraw, utab2, itab2, w_flat, b_vec)
    return out.reshape(_B, 1)


# speedup vs baseline: 1.0562x; 1.0562x over previous
import jax
import jax.numpy as jnp
from jax import lax
from jax.experimental import pallas as pl
from jax.experimental.pallas import tpu as pltpu
from jax.experimental.pallas import tpu_sc as plsc

_INFO = plsc.get_sparse_core_info()
_NC, _NS, _L = _INFO.num_cores, _INFO.num_subcores, _INFO.num_lanes
_NW = _NC * _NS

_B = 16384
_D = 64
_V = 1000000
_VMAIN = 999936
_NQ = _VMAIN // 512
_CPW = 61
_PITCH = 517
_LCAP = 800
_CCAP = 64
_DUMP = _B

_MESH = plsc.VectorSubcoreMesh(core_axis_name="c", subcore_axis_name="s")


def _wid():
    return lax.axis_index("s") * _NC + lax.axis_index("c")


def _ranges(wid):
    nq = _CPW + jnp.where(wid == 0, 1, 0)
    q0 = _CPW * wid + jnp.minimum(wid, 1)
    lo = q0 * 512
    hi = jnp.where(wid == _NW - 1, _V, lo + nq * 512)
    return q0, nq, lo, hi


def _prefilter(idx_vm, lidx, lpos, lo, hi, iota16):
    def init_body(v, c):
        lidx[pl.ds(v * _L, _L)] = jnp.full((_L,), -1, jnp.int32)
        lpos[pl.ds(v * _L, _L)] = _DUMP + iota16
        return c

    lax.fori_loop(0, _LCAP // _L, init_body, 0)

    def scan_body(v, off):
        iv = idx_vm[pl.ds(v * _L, _L)]
        pv = iota16 + v * _L
        m = (iv >= lo) & (iv < hi)
        plsc.store_compressed(lidx.at[pl.ds(off, _L)], iv, mask=m)
        plsc.store_compressed(lpos.at[pl.ds(off, _L)], pv, mask=m)
        return off + plsc.all_reduce_population_count(m)[0]

    return lax.fori_loop(0, _B // _L, scan_body, 0)


def _rescan(lidx, lpos, c0, c1, crm, cpos, iota16):
    def init_body(v, c):
        crm[pl.ds(v * _L, _L)] = jnp.zeros((_L,), jnp.int32)
        cpos[pl.ds(v * _L, _L)] = _DUMP + iota16
        return c

    lax.fori_loop(0, _CCAP // _L, init_body, 0)

    def scan_body(v, off):
        iv = lidx[pl.ds(v * _L, _L)]
        pv = lpos[pl.ds(v * _L, _L)]
        m = (iv >= c0) & (iv < c1)
        plsc.store_compressed(crm.at[pl.ds(off, _L)], iv - c0, mask=m)
        plsc.store_compressed(cpos.at[pl.ds(off, _L)], pv, mask=m)
        return off + plsc.all_reduce_population_count(m)[0]

    return lax.fori_loop(0, _LCAP // _L, scan_body, 0)


def _stage_chunk(tabT_hbm, tail_hbm, stage, col0, is_tail, sem):
    @pl.when(jnp.logical_not(is_tail))
    def _():
        cps = []
        for g in range(8):
            cps.append(pltpu.async_copy(
                tabT_hbm.at[pl.ds(g * 8, 8), pl.ds(col0, 512)],
                stage.at[pl.ds(g * 8, 8), pl.ds(0, 512)], sem))
        for c in cps:
            c.wait()

    @pl.when(is_tail)
    def _():
        pltpu.async_copy(tail_hbm, stage.at[:, pl.ds(0, 128)], sem).wait()


def _phase_a_body(uidx_hbm, utabT_hbm, tailu_hbm, w_hbm,
                  ustage_hbm,
                  idx_vm, w_v, stage, lidx, lpos, crm, cpos, obuf, opos, sem):
    wid = _wid()
    iota16 = lax.iota(jnp.int32, _L)
    q0, nq, lo, hi = _ranges(wid)

    pltpu.sync_copy(uidx_hbm, idx_vm)
    pltpu.sync_copy(w_hbm, w_v)
    wch = [w_v[pl.ds(ci * _L, _L)] for ci in range(4)]

    _prefilter(idx_vm, lidx, lpos, lo, hi, iota16)

    def chunk_body(c, carry):
        is_tail = (wid == _NW - 1) & (c == nq)
        active = (c < nq) | is_tail

        @pl.when(active)
        def _():
            col0 = (q0 + c) * 512
            c0 = jnp.where(is_tail, _VMAIN, col0)
            c1 = jnp.where(is_tail, _V, col0 + 512)
            _stage_chunk(utabT_hbm, tailu_hbm, stage, col0, is_tail, sem)
            nc = _rescan(lidx, lpos, c0, c1, crm, cpos, iota16)

            def elem_body(e, c2):
                rm = crm[pl.ds(e, _L)][0]
                for ci in range(4):
                    rows = iota16 + ci * _L
                    col = jnp.full((_L,), 0, jnp.int32) + rm
                    vals = plsc.load_gather(stage, [rows, col])
                    obuf[e, pl.ds(ci * _L, _L)] = vals * wch[ci]
                return c2

            lax.fori_loop(0, nc, elem_body, 0)
            for v in range(_CCAP // _L):
                opos[pl.ds(v * _L, _L)] = cpos[pl.ds(v * _L, _L)]
            pltpu.async_copy(obuf, ustage_hbm.at[opos], sem).wait()

        return carry

    lax.fori_loop(0, _CPW + 2, chunk_body, 0)


def _phase_b_body(iidx_hbm, itabT_hbm, taili_hbm, ustage_hbm, b_hbm,
                  vals_hbm, poss_hbm,
                  idx_vm, b_v, stage, lidx, lpos, crm, cpos, gpos, ugath,
                  accbuf, plist, vbuf, sem):
    wid = _wid()
    iota16 = lax.iota(jnp.int32, _L)
    q0, nq, lo, hi = _ranges(wid)

    pltpu.sync_copy(iidx_hbm, idx_vm)
    pltpu.sync_copy(b_hbm, b_v)
    bias = b_v[...]

    _prefilter(idx_vm, lidx, lpos, lo, hi, iota16)

    def init_plist(v, c):
        plist[pl.ds(v * _L, _L)] = _DUMP + iota16
        return c

    lax.fori_loop(0, _LCAP // _L, init_plist, 0)

    def chunk_body(c, lc0):
        is_tail = (wid == _NW - 1) & (c == nq)
        active = (c < nq) | is_tail

        def do_chunk(lc0):
            col0 = (q0 + c) * 512
            c0 = jnp.where(is_tail, _VMAIN, col0)
            c1 = jnp.where(is_tail, _V, col0 + 512)
            _stage_chunk(itabT_hbm, taili_hbm, stage, col0, is_tail, sem)
            nc = _rescan(lidx, lpos, c0, c1, crm, cpos, iota16)
            for v in range(_CCAP // _L):
                gpos[pl.ds(v * _L, _L)] = cpos[pl.ds(v * _L, _L)]
            pltpu.async_copy(ustage_hbm.at[gpos], ugath, sem).wait()

            def elem_body(e, c2):
                rm = crm[pl.ds(e, _L)][0]
                acc = jnp.zeros((_L,), jnp.float32)
                for ci in range(4):
                    rows = iota16 + ci * _L
                    col = jnp.full((_L,), 0, jnp.int32) + rm
                    iv = plsc.load_gather(stage, [rows, col])
                    acc = acc + iv * ugath[e, pl.ds(ci * _L, _L)]
                accbuf[pl.ds((lc0 + e) * 17, _L)] = acc
                return c2

            lax.fori_loop(0, nc, elem_body, 0)
            for v in range(_CCAP // _L):
                plist[pl.ds(lc0 + v * _L, _L)] = cpos[pl.ds(v * _L, _L)]
            return lc0 + nc

        return lax.cond(active, do_chunk, lambda x: x, lc0)

    lax.fori_loop(0, _CPW + 2, chunk_body, 0)

    def grp_body(g, carry):
        flat = iota16 * 17 + g * (_L * 17)
        acc = bias
        for l in range(_L):
            acc = acc + plsc.load_gather(accbuf, [flat + l])
        vbuf[pl.ds(g * _L, _L)] = 1.0 / (1.0 + jnp.exp(-acc))
        return carry

    lax.fori_loop(0, _LCAP // _L, grp_body, 0)
    pltpu.sync_copy(vbuf, vals_hbm.at[pl.ds(wid * _LCAP, _LCAP)])
    pltpu.sync_copy(plist.at[pl.ds(0, _LCAP)],
                    poss_hbm.at[pl.ds(wid * _LCAP, _LCAP)])


def _phase_c_body(vals_hbm, poss_hbm, out_hbm, vals_vm, poss_vm, sbuf, sem):
    wid = _wid()
    pltpu.sync_copy(vals_hbm, vals_vm)
    pltpu.sync_copy(poss_hbm, poss_vm)

    def scat_body(v, c):
        pv = poss_vm[pl.ds(v * _L, _L)]
        vv = vals_vm[pl.ds(v * _L, _L)]
        plsc.store_scatter(sbuf, [pv], vv)
        return c

    lax.fori_loop(0, (_NW * _LCAP) // _L, scat_body, 0)
    base = wid * (_B // _NW)
    pltpu.sync_copy(sbuf.at[pl.ds(base, _B // _NW)],
                    out_hbm.at[pl.ds(base, _B // _NW)])


@jax.jit
def _gmf_call(uidx, iidx, utabT, itabT, tailu, taili, w_flat, b_vec):
    ustage = pl.kernel(
        _phase_a_body,
        mesh=_MESH,
        out_type=jax.ShapeDtypeStruct((_B + _L, 128), jnp.float32),
        scratch_types=[
            pltpu.VMEM((_B,), jnp.int32),
            pltpu.VMEM((_D,), jnp.float32),
            pltpu.VMEM((_D, _PITCH), jnp.float32),
            pltpu.VMEM((_LCAP,), jnp.int32),
            pltpu.VMEM((_LCAP,), jnp.int32),
            pltpu.VMEM((_CCAP + _L,), jnp.int32),
            pltpu.VMEM((_CCAP + _L,), jnp.int32),
            pltpu.VMEM((_CCAP, 128), jnp.float32),
            pltpu.VMEM((_CCAP,), jnp.int32),
            pltpu.SemaphoreType.DMA,
        ],
        compiler_params=pltpu.CompilerParams(needs_layout_passes=False),
    )(uidx, utabT, tailu, w_flat)

    vals, poss = pl.kernel(
        _phase_b_body,
        mesh=_MESH,
        out_type=(jax.ShapeDtypeStruct((_NW * _LCAP,), jnp.float32),
                  jax.ShapeDtypeStruct((_NW * _LCAP,), jnp.int32)),
        scratch_types=[
            pltpu.VMEM((_B,), jnp.int32),
            pltpu.VMEM((_L,), jnp.float32),
            pltpu.VMEM((_D, _PITCH), jnp.float32),
            pltpu.VMEM((_LCAP,), jnp.int32),
            pltpu.VMEM((_LCAP,), jnp.int32),
            pltpu.VMEM((_CCAP + _L,), jnp.int32),
            pltpu.VMEM((_CCAP + _L,), jnp.int32),
            pltpu.VMEM((_CCAP,), jnp.int32),
            pltpu.VMEM((_CCAP, 128), jnp.float32),
            pltpu.VMEM(((_LCAP + _CCAP) * 17,), jnp.float32),
            pltpu.VMEM((_LCAP + _CCAP,), jnp.int32),
            pltpu.VMEM((_LCAP,), jnp.float32),
            pltpu.SemaphoreType.DMA,
        ],
        compiler_params=pltpu.CompilerParams(needs_layout_passes=False),
    )(iidx, itabT, taili, ustage, b_vec)

    out = pl.kernel(
        _phase_c_body,
        mesh=_MESH,
        out_type=jax.ShapeDtypeStruct((_B,), jnp.float32),
        scratch_types=[
            pltpu.VMEM((_NW * _LCAP,), jnp.float32),
            pltpu.VMEM((_NW * _LCAP,), jnp.int32),
            pltpu.VMEM((_B + 2 * _L,), jnp.float32),
            pltpu.SemaphoreType.DMA,
        ],
        compiler_params=pltpu.CompilerParams(needs_layout_passes=False),
    )(vals, poss)
    return out


def kernel(user_indices, item_indices, user_table, item_table, affine_w,
           affine_b):
    uidx = user_indices.astype(jnp.int32)
    iidx = item_indices.astype(jnp.int32)
    utabT = user_table.T
    itabT = item_table.T
    tailu = jnp.pad(utabT[:, _VMAIN:], ((0, 0), (0, 128 - (_V - _VMAIN))))
    taili = jnp.pad(itabT[:, _VMAIN:], ((0, 0), (0, 128 - (_V - _VMAIN))))
    w_flat = affine_w.reshape(_D)
    b_vec = jnp.broadcast_to(affine_b.reshape(()), (_L,))
    out = _gmf_call(uidx, iidx, utabT, itabT, tailu, taili, w_flat, b_vec)
    return out.reshape(_B, 1)


# speedup vs baseline: 1.0607x; 1.0043x over previous
import jax
import jax.numpy as jnp
from jax import lax
from jax.experimental import pallas as pl
from jax.experimental.pallas import tpu as pltpu
from jax.experimental.pallas import tpu_sc as plsc

_INFO = plsc.get_sparse_core_info()
_NC, _NS, _L = _INFO.num_cores, _INFO.num_subcores, _INFO.num_lanes
_NW = _NC * _NS

_B = 16384
_D = 64
_V = 1000000
_VMAIN = 999936
_NQ = _VMAIN // 512
_CPW = 61
_PITCH = 517
_LCAP = 800
_CCAP = 64
_DUMP = _B

_MESH = plsc.VectorSubcoreMesh(core_axis_name="c", subcore_axis_name="s")


def _wid():
    return lax.axis_index("s") * _NC + lax.axis_index("c")


def _ranges(wid):
    nq = _CPW + jnp.where(wid == 0, 1, 0)
    q0 = _CPW * wid + jnp.minimum(wid, 1)
    lo = q0 * 512
    hi = jnp.where(wid == _NW - 1, _V, lo + nq * 512)
    return q0, nq, lo, hi


def _prefilter(idx_vm, lidx, lpos, lo, hi, iota16):
    def init_body(v, c):
        lidx[pl.ds(v * _L, _L)] = jnp.full((_L,), -1, jnp.int32)
        lpos[pl.ds(v * _L, _L)] = _DUMP + iota16
        return c

    lax.fori_loop(0, _LCAP // _L, init_body, 0)

    def scan_body(v, off):
        iv = idx_vm[pl.ds(v * _L, _L)]
        pv = iota16 + v * _L
        m = (iv >= lo) & (iv < hi)
        plsc.store_compressed(lidx.at[pl.ds(off, _L)], iv, mask=m)
        plsc.store_compressed(lpos.at[pl.ds(off, _L)], pv, mask=m)
        return off + plsc.all_reduce_population_count(m)[0]

    return lax.fori_loop(0, _B // _L, scan_body, 0)


def _rescan(lidx, lpos, c0, c1, crm, cpos, iota16):
    def init_body(v, c):
        crm[pl.ds(v * _L, _L)] = jnp.zeros((_L,), jnp.int32)
        cpos[pl.ds(v * _L, _L)] = _DUMP + iota16
        return c

    lax.fori_loop(0, _CCAP // _L, init_body, 0)

    def scan_body(v, off):
        iv = lidx[pl.ds(v * _L, _L)]
        pv = lpos[pl.ds(v * _L, _L)]
        m = (iv >= c0) & (iv < c1)
        plsc.store_compressed(crm.at[pl.ds(off, _L)], iv - c0, mask=m)
        plsc.store_compressed(cpos.at[pl.ds(off, _L)], pv, mask=m)
        return off + plsc.all_reduce_population_count(m)[0]

    return lax.fori_loop(0, _LCAP // _L, scan_body, 0)


def _fire_chunk(tabT_hbm, tail_hbm, stage, slot, q0, c, nq, wid, sem):
    is_tail = (wid == _NW - 1) & (c == nq)
    col0 = (q0 + c) * 512

    @pl.when(c < nq)
    def _():
        for g in range(8):
            pltpu.async_copy(
                tabT_hbm.at[pl.ds(g * 8, 8), pl.ds(col0, 512)],
                stage.at[slot, pl.ds(g * 8, 8), pl.ds(0, 512)], sem)

    @pl.when(is_tail)
    def _():
        pltpu.async_copy(tail_hbm, stage.at[slot, :, pl.ds(0, 128)], sem)


def _drain_chunk(tabT_hbm, tail_hbm, stage, slot, c, nq, wid, sem):
    is_tail = (wid == _NW - 1) & (c == nq)

    @pl.when(c < nq)
    def _():
        for g in range(8):
            pltpu.make_async_copy(
                tabT_hbm.at[pl.ds(g * 8, 8), pl.ds(0, 512)],
                stage.at[slot, pl.ds(g * 8, 8), pl.ds(0, 512)], sem).wait()

    @pl.when(is_tail)
    def _():
        pltpu.make_async_copy(
            tail_hbm, stage.at[slot, :, pl.ds(0, 128)], sem).wait()


def _phase_a_body(uidx_hbm, utabT_hbm, tailu_hbm, w_hbm,
                  ustage_hbm,
                  idx_vm, w_v, stage, lidx, lpos, crm, cpos, obuf, opos, sem, osem):
    wid = _wid()
    iota16 = lax.iota(jnp.int32, _L)
    q0, nq, lo, hi = _ranges(wid)

    pltpu.sync_copy(uidx_hbm, idx_vm)
    pltpu.sync_copy(w_hbm, w_v)
    wch = [w_v[pl.ds(ci * _L, _L)] for ci in range(4)]

    _prefilter(idx_vm, lidx, lpos, lo, hi, iota16)

    _fire_chunk(utabT_hbm, tailu_hbm, stage, 0, q0, 0, nq, wid, sem)

    def chunk_body(c, carry):
        is_tail = (wid == _NW - 1) & (c == nq)
        active = (c < nq) | is_tail
        slot = lax.rem(c, 2)

        _fire_chunk(utabT_hbm, tailu_hbm, stage, 1 - slot, q0, c + 1, nq,
                    wid, sem)
        _drain_chunk(utabT_hbm, tailu_hbm, stage, slot, c, nq, wid, sem)

        @pl.when(active)
        def _():
            col0 = (q0 + c) * 512
            c0 = jnp.where(is_tail, _VMAIN, col0)
            c1 = jnp.where(is_tail, _V, col0 + 512)
            nc = _rescan(lidx, lpos, c0, c1, crm, cpos, iota16)

            def elem_body(e, c2):
                rm = crm[pl.ds(e, _L)][0]
                for ci in range(4):
                    rows = iota16 + ci * _L
                    col = jnp.full((_L,), 0, jnp.int32) + rm
                    vals = plsc.load_gather(stage.at[slot], [rows, col])
                    obuf[e, pl.ds(ci * _L, _L)] = vals * wch[ci]
                return c2

            lax.fori_loop(0, nc, elem_body, 0)
            for v in range(_CCAP // _L):
                opos[pl.ds(v * _L, _L)] = cpos[pl.ds(v * _L, _L)]
            pltpu.async_copy(obuf, ustage_hbm.at[opos], osem).wait()

        return carry

    lax.fori_loop(0, _CPW + 2, chunk_body, 0)


def _phase_b_body(iidx_hbm, itabT_hbm, taili_hbm, ustage_hbm, b_hbm,
                  vals_hbm, poss_hbm,
                  idx_vm, b_v, stage, lidx, lpos, crm, cpos, gpos, ugath,
                  accbuf, plist, vbuf, sem, osem):
    wid = _wid()
    iota16 = lax.iota(jnp.int32, _L)
    q0, nq, lo, hi = _ranges(wid)

    pltpu.sync_copy(iidx_hbm, idx_vm)
    pltpu.sync_copy(b_hbm, b_v)
    bias = b_v[...]

    _prefilter(idx_vm, lidx, lpos, lo, hi, iota16)

    def init_plist(v, c):
        plist[pl.ds(v * _L, _L)] = _DUMP + iota16
        return c

    lax.fori_loop(0, _LCAP // _L, init_plist, 0)

    _fire_chunk(itabT_hbm, taili_hbm, stage, 0, q0, 0, nq, wid, sem)

    def chunk_body(c, lc0):
        is_tail = (wid == _NW - 1) & (c == nq)
        active = (c < nq) | is_tail
        slot = lax.rem(c, 2)

        _fire_chunk(itabT_hbm, taili_hbm, stage, 1 - slot, q0, c + 1, nq,
                    wid, sem)
        _drain_chunk(itabT_hbm, taili_hbm, stage, slot, c, nq, wid, sem)

        def do_chunk(lc0):
            col0 = (q0 + c) * 512
            c0 = jnp.where(is_tail, _VMAIN, col0)
            c1 = jnp.where(is_tail, _V, col0 + 512)
            nc = _rescan(lidx, lpos, c0, c1, crm, cpos, iota16)
            for v in range(_CCAP // _L):
                gpos[pl.ds(v * _L, _L)] = cpos[pl.ds(v * _L, _L)]
            pltpu.async_copy(ustage_hbm.at[gpos], ugath, osem).wait()

            def elem_body(e, c2):
                rm = crm[pl.ds(e, _L)][0]
                acc = jnp.zeros((_L,), jnp.float32)
                for ci in range(4):
                    rows = iota16 + ci * _L
                    col = jnp.full((_L,), 0, jnp.int32) + rm
                    iv = plsc.load_gather(stage.at[slot], [rows, col])
                    acc = acc + iv * ugath[e, pl.ds(ci * _L, _L)]
                accbuf[pl.ds((lc0 + e) * 17, _L)] = acc
                return c2

            lax.fori_loop(0, nc, elem_body, 0)
            for v in range(_CCAP // _L):
                plist[pl.ds(lc0 + v * _L, _L)] = cpos[pl.ds(v * _L, _L)]
            return lc0 + nc

        return lax.cond(active, do_chunk, lambda x: x, lc0)

    lax.fori_loop(0, _CPW + 2, chunk_body, 0)

    def grp_body(g, carry):
        flat = iota16 * 17 + g * (_L * 17)
        acc = bias
        for l in range(_L):
            acc = acc + plsc.load_gather(accbuf, [flat + l])
        vbuf[pl.ds(g * _L, _L)] = 1.0 / (1.0 + jnp.exp(-acc))
        return carry

    lax.fori_loop(0, _LCAP // _L, grp_body, 0)
    pltpu.sync_copy(vbuf, vals_hbm.at[pl.ds(wid * _LCAP, _LCAP)])
    pltpu.sync_copy(plist.at[pl.ds(0, _LCAP)],
                    poss_hbm.at[pl.ds(wid * _LCAP, _LCAP)])


def _phase_c_body(vals_hbm, poss_hbm, out_hbm, vals_vm, poss_vm, sbuf, sem):
    wid = _wid()
    pltpu.sync_copy(vals_hbm, vals_vm)
    pltpu.sync_copy(poss_hbm, poss_vm)

    def scat_body(v, c):
        pv = poss_vm[pl.ds(v * _L, _L)]
        vv = vals_vm[pl.ds(v * _L, _L)]
        plsc.store_scatter(sbuf, [pv], vv)
        return c

    lax.fori_loop(0, (_NW * _LCAP) // _L, scat_body, 0)
    base = wid * (_B // _NW)
    pltpu.sync_copy(sbuf.at[pl.ds(base, _B // _NW)],
                    out_hbm.at[pl.ds(base, _B // _NW)])


@jax.jit
def _gmf_call(uidx, iidx, utabT, itabT, tailu, taili, w_flat, b_vec):
    ustage = pl.kernel(
        _phase_a_body,
        mesh=_MESH,
        out_type=jax.ShapeDtypeStruct((_B + _L, 128), jnp.float32),
        scratch_types=[
            pltpu.VMEM((_B,), jnp.int32),
            pltpu.VMEM((_D,), jnp.float32),
            pltpu.VMEM((2, _D, _PITCH), jnp.float32),
            pltpu.VMEM((_LCAP,), jnp.int32),
            pltpu.VMEM((_LCAP,), jnp.int32),
            pltpu.VMEM((_CCAP + _L,), jnp.int32),
            pltpu.VMEM((_CCAP + _L,), jnp.int32),
            pltpu.VMEM((_CCAP, 128), jnp.float32),
            pltpu.VMEM((_CCAP,), jnp.int32),
            pltpu.SemaphoreType.DMA,
            pltpu.SemaphoreType.DMA,
        ],
        compiler_params=pltpu.CompilerParams(needs_layout_passes=False),
    )(uidx, utabT, tailu, w_flat)

    vals, poss = pl.kernel(
        _phase_b_body,
        mesh=_MESH,
        out_type=(jax.ShapeDtypeStruct((_NW * _LCAP,), jnp.float32),
                  jax.ShapeDtypeStruct((_NW * _LCAP,), jnp.int32)),
        scratch_types=[
            pltpu.VMEM((_B,), jnp.int32),
            pltpu.VMEM((_L,), jnp.float32),
            pltpu.VMEM((2, _D, _PITCH), jnp.float32),
            pltpu.VMEM((_LCAP,), jnp.int32),
            pltpu.VMEM((_LCAP,), jnp.int32),
            pltpu.VMEM((_CCAP + _L,), jnp.int32),
            pltpu.VMEM((_CCAP + _L,), jnp.int32),
            pltpu.VMEM((_CCAP,), jnp.int32),
            pltpu.VMEM((_CCAP, 128), jnp.float32),
            pltpu.VMEM(((_LCAP + _CCAP) * 17,), jnp.float32),
            pltpu.VMEM((_LCAP + _CCAP,), jnp.int32),
            pltpu.VMEM((_LCAP,), jnp.float32),
            pltpu.SemaphoreType.DMA,
            pltpu.SemaphoreType.DMA,
        ],
        compiler_params=pltpu.CompilerParams(needs_layout_passes=False),
    )(iidx, itabT, taili, ustage, b_vec)

    out = pl.kernel(
        _phase_c_body,
        mesh=_MESH,
        out_type=jax.ShapeDtypeStruct((_B,), jnp.float32),
        scratch_types=[
            pltpu.VMEM((_NW * _LCAP,), jnp.float32),
            pltpu.VMEM((_NW * _LCAP,), jnp.int32),
            pltpu.VMEM((_B + 2 * _L,), jnp.float32),
            pltpu.SemaphoreType.DMA,
        ],
        compiler_params=pltpu.CompilerParams(needs_layout_passes=False),
    )(vals, poss)
    return out


def kernel(user_indices, item_indices, user_table, item_table, affine_w,
           affine_b):
    uidx = user_indices.astype(jnp.int32)
    iidx = item_indices.astype(jnp.int32)
    utabT = user_table.T
    itabT = item_table.T
    tailu = jnp.pad(utabT[:, _VMAIN:], ((0, 0), (0, 128 - (_V - _VMAIN))))
    taili = jnp.pad(itabT[:, _VMAIN:], ((0, 0), (0, 128 - (_V - _VMAIN))))
    w_flat = affine_w.reshape(_D)
    b_vec = jnp.broadcast_to(affine_b.reshape(()), (_L,))
    out = _gmf_call(uidx, iidx, utabT, itabT, tailu, taili, w_flat, b_vec)
    return out.reshape(_B, 1)


# speedup vs baseline: 2.1109x; 1.9902x over previous
import jax
import jax.numpy as jnp
from jax import lax
from jax.experimental import pallas as pl
from jax.experimental.pallas import tpu as pltpu
from jax.experimental.pallas import tpu_sc as plsc

_INFO = plsc.get_sparse_core_info()
_NC, _NS, _L = _INFO.num_cores, _INFO.num_subcores, _INFO.num_lanes
_NW = _NC * _NS

_B = 16384
_D = 64
_V = 1000000
_CW = 1024
_NCH = 61
_VMAIN = 16 * _NCH * _CW
_TAILW = 640
_PITCH = 1029
_LCAP = 1664
_CCAP = 64
_DUMP = _B
_BPW = _B // _NW

_MESH = plsc.VectorSubcoreMesh(core_axis_name="c", subcore_axis_name="s")


def _prefilter(idx_vm, lidx, lpos, lo, hi, iota16):
    def init_body(v, c):
        lidx[pl.ds(v * _L, _L)] = jnp.full((_L,), -1, jnp.int32)
        lpos[pl.ds(v * _L, _L)] = _DUMP + iota16
        return c

    lax.fori_loop(0, _LCAP // _L, init_body, 0, unroll=8)

    def scan_body(v, off):
        iv = idx_vm[pl.ds(v * _L, _L)]
        pv = iota16 + v * _L
        m = (iv >= lo) & (iv < hi)
        plsc.store_compressed(lidx.at[pl.ds(off, _L)], iv, mask=m)
        plsc.store_compressed(lpos.at[pl.ds(off, _L)], pv, mask=m)
        return off + plsc.all_reduce_population_count(m)[0]

    return lax.fori_loop(0, _B // _L, scan_body, 0, unroll=8)


def _rescan(lidx, lpos, c0, c1, crm, cpos, iota16):
    def init_body(v, c):
        crm[pl.ds(v * _L, _L)] = jnp.zeros((_L,), jnp.int32)
        cpos[pl.ds(v * _L, _L)] = _DUMP + iota16
        return c

    lax.fori_loop(0, _CCAP // _L, init_body, 0, unroll=4)

    def scan_body(v, off):
        iv = lidx[pl.ds(v * _L, _L)]
        pv = lpos[pl.ds(v * _L, _L)]
        m = (iv >= c0) & (iv < c1)
        plsc.store_compressed(crm.at[pl.ds(off, _L)], iv - c0, mask=m)
        plsc.store_compressed(cpos.at[pl.ds(off, _L)], pv, mask=m)
        return off + plsc.all_reduce_population_count(m)[0]

    return lax.fori_loop(0, _LCAP // _L, scan_body, 0, unroll=8)


def _scan_table(idx_hbm, tabT_hbm, tail_hbm, stage_hbm, s, scale, wch,
                idx_vm, stage, lidx, lpos, crm, cpos, obuf, opos, sem, osem,
                iota16):
    lo = s * (_NCH * _CW)
    hi = jnp.where(s == _NS - 1, _V, lo + _NCH * _CW)

    pltpu.sync_copy(idx_hbm, idx_vm)
    _prefilter(idx_vm, lidx, lpos, lo, hi, iota16)

    def chunk_body(c, carry):
        is_tail = (s == _NS - 1) & (c == _NCH)
        active = (c < _NCH) | is_tail
        slot = lax.rem(c, 2)

        @pl.when(active)
        def _():
            col0 = lo + c * _CW
            c0 = jnp.where(is_tail, _VMAIN, col0)
            c1 = jnp.where(is_tail, _V, col0 + _CW)

            @pl.when(jnp.logical_not(is_tail))
            def _():
                cps = []
                for g in range(8):
                    cps.append(pltpu.async_copy(
                        tabT_hbm.at[pl.ds(g * 8, 8), pl.ds(col0, _CW)],
                        stage.at[pl.ds(g * 8, 8), pl.ds(0, _CW)], sem))
                for cp in cps:
                    cp.wait()

            @pl.when(is_tail)
            def _():
                pltpu.async_copy(tail_hbm, stage.at[:, pl.ds(0, _TAILW)],
                                 sem).wait()

            nc = _rescan(lidx, lpos, c0, c1, crm, cpos, iota16)

            @pl.when(c >= 2)
            def _():
                pltpu.make_async_copy(
                    obuf.at[slot], stage_hbm.at[opos.at[slot]], osem).wait()

            def elem_body(e, c2):
                rm = crm[pl.ds(e, _L)][0]
                for ci in range(4):
                    rows = iota16 + ci * _L
                    col = jnp.full((_L,), 0, jnp.int32) + rm
                    vals = plsc.load_gather(stage, [rows, col])
                    if scale:
                        vals = vals * wch[ci]
                    obuf[slot, e, pl.ds(ci * _L, _L)] = vals
                return c2

            lax.fori_loop(0, nc, elem_body, 0)
            for v in range(_CCAP // _L):
                opos[slot, pl.ds(v * _L, _L)] = cpos[pl.ds(v * _L, _L)]
            pltpu.async_copy(obuf.at[slot], stage_hbm.at[opos.at[slot]], osem)

        return carry

    lax.fori_loop(0, _NCH + 1, chunk_body, 0)
    for slot in range(2):
        pltpu.make_async_copy(obuf.at[slot], stage_hbm.at[opos.at[slot]],
                              osem).wait()


def _scan_body(uidx_hbm, iidx_hbm, utabT_hbm, itabT_hbm, tailu_hbm,
               taili_hbm, w_hbm,
               ustage_hbm, istage_hbm,
               idx_vm, w_v, stage, lidx, lpos, crm, cpos, obuf, opos,
               sem, osem):
    sc = lax.axis_index("c")
    s = lax.axis_index("s")
    iota16 = lax.iota(jnp.int32, _L)
    pltpu.sync_copy(w_hbm, w_v)
    wch = [w_v[pl.ds(ci * _L, _L)] for ci in range(4)]

    @pl.when(sc == 0)
    def _():
        _scan_table(uidx_hbm, utabT_hbm, tailu_hbm, ustage_hbm, s, True, wch,
                    idx_vm, stage, lidx, lpos, crm, cpos, obuf, opos,
                    sem, osem, iota16)

    @pl.when(sc == 1)
    def _():
        _scan_table(iidx_hbm, itabT_hbm, taili_hbm, istage_hbm, s, False, wch,
                    idx_vm, stage, lidx, lpos, crm, cpos, obuf, opos,
                    sem, osem, iota16)


def _combine_body(ustage_hbm, istage_hbm, b_hbm, out_hbm,
                  ubuf, ibuf, b_v, accbuf, outbuf, sem):
    wid = lax.axis_index("s") * _NC + lax.axis_index("c")
    base = wid * _BPW
    iota16 = lax.iota(jnp.int32, _L)
    pltpu.sync_copy(b_hbm, b_v)
    bias = b_v[...]

    def quarter_body(q, carry):
        rb = base + q * 128
        pltpu.sync_copy(ustage_hbm.at[pl.ds(rb, 128)], ubuf)
        pltpu.sync_copy(istage_hbm.at[pl.ds(rb, 128)], ibuf)

        def row_body(k, c2):
            acc = (ubuf[k, pl.ds(0, _L)] * ibuf[k, pl.ds(0, _L)]
                   + ubuf[k, pl.ds(_L, _L)] * ibuf[k, pl.ds(_L, _L)]
                   + ubuf[k, pl.ds(2 * _L, _L)] * ibuf[k, pl.ds(2 * _L, _L)]
                   + ubuf[k, pl.ds(3 * _L, _L)] * ibuf[k, pl.ds(3 * _L, _L)])
            accbuf[pl.ds((q * 128 + k) * 17, _L)] = acc
            return c2

        lax.fori_loop(0, 128, row_body, 0, unroll=4)
        return carry

    lax.fori_loop(0, _BPW // 128, quarter_body, 0)

    def grp_body(g, carry):
        flat = iota16 * 17 + g * (_L * 17)
        acc = bias
        for l in range(_L):
            acc = acc + plsc.load_gather(accbuf, [flat + l])
        outbuf[pl.ds(g * _L, _L)] = 1.0 / (1.0 + jnp.exp(-acc))
        return carry

    lax.fori_loop(0, _BPW // _L, grp_body, 0)
    pltpu.sync_copy(outbuf, out_hbm.at[pl.ds(base, _BPW)])


@jax.jit
def _gmf_call(uidx, iidx, utabT, itabT, tailu, taili, w_flat, b_vec):
    ustage, istage = pl.kernel(
        _scan_body,
        mesh=_MESH,
        out_type=(jax.ShapeDtypeStruct((_B + _L, 128), jnp.float32),
                  jax.ShapeDtypeStruct((_B + _L, 128), jnp.float32)),
        scratch_types=[
            pltpu.VMEM((_B,), jnp.int32),
            pltpu.VMEM((_D,), jnp.float32),
            pltpu.VMEM((_D, _PITCH), jnp.float32),
            pltpu.VMEM((_LCAP,), jnp.int32),
            pltpu.VMEM((_LCAP,), jnp.int32),
            pltpu.VMEM((_CCAP + _L,), jnp.int32),
            pltpu.VMEM((_CCAP + _L,), jnp.int32),
            pltpu.VMEM((2, _CCAP, 128), jnp.float32),
            pltpu.VMEM((2, _CCAP), jnp.int32),
            pltpu.SemaphoreType.DMA,
            pltpu.SemaphoreType.DMA,
        ],
        compiler_params=pltpu.CompilerParams(needs_layout_passes=False),
    )(uidx, iidx, utabT, itabT, tailu, taili, w_flat)

    out = pl.kernel(
        _combine_body,
        mesh=_MESH,
        out_type=jax.ShapeDtypeStruct((_B,), jnp.float32),
        scratch_types=[
            pltpu.VMEM((128, 128), jnp.float32),
            pltpu.VMEM((128, 128), jnp.float32),
            pltpu.VMEM((_L,), jnp.float32),
            pltpu.VMEM((_BPW * 17,), jnp.float32),
            pltpu.VMEM((_BPW,), jnp.float32),
            pltpu.SemaphoreType.DMA,
        ],
        compiler_params=pltpu.CompilerParams(needs_layout_passes=False),
    )(ustage, istage, b_vec)
    return out


def kernel(user_indices, item_indices, user_table, item_table, affine_w,
           affine_b):
    uidx = user_indices.astype(jnp.int32)
    iidx = item_indices.astype(jnp.int32)
    utabT = user_table.T
    itabT = item_table.T
    tailu = jnp.pad(utabT[:, _VMAIN:], ((0, 0), (0, _TAILW - (_V - _VMAIN))))
    taili = jnp.pad(itabT[:, _VMAIN:], ((0, 0), (0, _TAILW - (_V - _VMAIN))))
    w_flat = affine_w.reshape(_D)
    b_vec = jnp.broadcast_to(affine_b.reshape(()), (_L,))
    out = _gmf_call(uidx, iidx, utabT, itabT, tailu, taili, w_flat, b_vec)
    return out.reshape(_B, 1)
